# Initial kernel scaffold; baseline (speedup 1.0000x reference)
import jax
import jax.numpy as jnp
from jax.experimental import pallas as pl


def kernel(x_clause, x_variable, deg_clause, deg_variable, ei_cp, ei_cn, ei_rp, ei_rn, W0c, W0v, conv_ls_W, conv_ls_b, mlp_W1, mlp_b1, mlp_W2, mlp_b2, mlp_W3, mlp_b3, lins_c_W, lins_c_b, lins_v_W, lins_v_b):
    raise NotImplementedError("write your pallas kernel here")


# speedup vs baseline: 13.0522x; 13.0522x over previous
import functools

import jax
import jax.numpy as jnp
from jax import lax
from jax.experimental import pallas as pl
from jax.experimental.pallas import tpu as pltpu
from jax.experimental.pallas import tpu_sc as plsc

H = 128
HM = 153
HMP = 160
N = 10000
E = 160000
F32 = jnp.float32

NCORES = 2
NSUB = 16
NW = NCORES * NSUB
CHUNK = 128
NCHUNKS = E // CHUNK
CPW = NCHUNKS // NW
CREM = NCHUNKS % NW
NP = 10240
RPT = NP // NSUB


def _inv_sqrt(d):
    safe = jnp.where(d > 0, d, 1.0)
    return jnp.where(d > 0, lax.rsqrt(safe), 0.0)


def _mlp_pair_body(rank1, x_ref, deg_ref, W1_ref, b1_ref, W2_ref, b2_ref,
                   W3_ref, b3_ref, oa_ref, ob_ref):
    dsi = _inv_sqrt(deg_ref[...])
    x = x_ref[...]
    for k, out in ((0, oa_ref), (1, ob_ref)):
        if rank1:
            h = x * W1_ref[k] + b1_ref[k]
        else:
            h = jnp.dot(x, W1_ref[k], preferred_element_type=F32) + b1_ref[k]
        h = jnp.maximum(h, 0.0)
        h = jnp.maximum(jnp.dot(h, W2_ref[k], preferred_element_type=F32) + b2_ref[k], 0.0)
        h = jnp.maximum(jnp.dot(h, W3_ref[k], preferred_element_type=F32) + b3_ref[k], 0.0)
        out[...] = h * dsi


def _mlp_pair(x, deg, W1, b1, W2, b2, W3, b3, rank1, br=1000):
    grid = (N // br,)
    kdim = 1 if rank1 else H
    full = lambda *s: pl.BlockSpec(s, lambda i: (0,) * len(s))
    out = pl.pallas_call(
        functools.partial(_mlp_pair_body, rank1),
        grid=grid,
        in_specs=[
            pl.BlockSpec((br, kdim), lambda i: (i, 0)),
            pl.BlockSpec((br, 1), lambda i: (i, 0)),
            full(2, kdim, HMP), full(2, 1, HMP),
            full(2, HMP, HMP), full(2, 1, HMP),
            full(2, HMP, H), full(2, 1, H),
        ],
        out_specs=[pl.BlockSpec((br, H), lambda i: (i, 0))] * 2,
        out_shape=[jax.ShapeDtypeStruct((N, H), F32)] * 2,
    )(x, deg, W1, b1, W2, b2, W3, b3)
    return out


def _combine_body(rank1, pp_ref, pn_ref, deg_ref, prev_ref, Wa_ref, Wb_ref,
                  Wc_ref, b_ref, out_ref):
    dti = _inv_sqrt(deg_ref[...])
    a = (pp_ref[0] + pp_ref[1]) * dti
    c = (pn_ref[0] + pn_ref[1]) * dti
    acc = jnp.dot(a, Wa_ref[...], preferred_element_type=F32)
    acc += jnp.dot(c, Wb_ref[...], preferred_element_type=F32)
    if rank1:
        acc += prev_ref[...] * Wc_ref[...]
    else:
        acc += jnp.dot(prev_ref[...], Wc_ref[...], preferred_element_type=F32)
    out_ref[...] = acc + b_ref[...]


def _combine(pp, pn, deg, prev, Wa, Wb, Wc, b, rank1, br=1000):
    grid = (N // br,)
    kdim = 1 if rank1 else H
    full = lambda *s: pl.BlockSpec(s, lambda i: (0,) * len(s))
    return pl.pallas_call(
        functools.partial(_combine_body, rank1),
        grid=grid,
        in_specs=[
            pl.BlockSpec((2, br, H), lambda i: (0, i, 0)),
            pl.BlockSpec((2, br, H), lambda i: (0, i, 0)),
            pl.BlockSpec((br, 1), lambda i: (i, 0)),
            pl.BlockSpec((br, kdim), lambda i: (i, 0)),
            full(H, H), full(H, H), full(kdim, H), full(1, H),
        ],
        out_specs=pl.BlockSpec((br, H), lambda i: (i, 0)),
        out_shape=jax.ShapeDtypeStruct((N, H), F32),
    )(pp, pn, deg, prev, Wa, Wb, Wc, b)


def _conv2_body(z0, s0, t0, z1, s1, t1, zrows, out0, out1,
                acc, idx_s, idx_t, rows, sem):
    cid = lax.axis_index("c")
    sid = lax.axis_index("s")
    wid = sid * NCORES + cid
    base = sid * RPT
    nj = CPW + jnp.where(wid < CREM, 1, 0)
    for (z, s, t, out) in ((z0, s0, t0, out0), (z1, s1, t1, out1)):
        pltpu.sync_copy(zrows, acc.at[pl.ds(base, RPT)])
        plsc.subcore_barrier()

        @pl.loop(0, nj)
        def _(j):
            off = (wid + NW * j) * CHUNK
            pltpu.sync_copy(s.at[pl.ds(off, CHUNK)], idx_s)
            pltpu.sync_copy(t.at[pl.ds(off, CHUNK)], idx_t)
            pltpu.async_copy(z.at[idx_s], rows, sem).wait()
            pltpu.sync_copy(rows, acc.at[idx_t], add=True)

        plsc.subcore_barrier()
        pltpu.sync_copy(acc.at[pl.ds(base, RPT)],
                        out.at[pl.ds(cid * NP + base, RPT)])


@functools.cache
def _conv2_kernel():
    mesh = plsc.VectorSubcoreMesh(core_axis_name="c", subcore_axis_name="s")
    return pl.kernel(
        _conv2_body,
        mesh=mesh,
        out_type=[jax.ShapeDtypeStruct((NCORES * NP, H), F32)] * 2,
        scratch_types=[
            pltpu.VMEM_SHARED((NP, H), F32),
            pltpu.VMEM((CHUNK,), jnp.int32),
            pltpu.VMEM((CHUNK,), jnp.int32),
            pltpu.VMEM((CHUNK, H), F32),
            pltpu.SemaphoreType.DMA,
        ],
    )


def _conv_pair(z0, ei0, z1, ei1, zrows):
    p0, p1 = _conv2_kernel()(z0, ei0[0], ei0[1], z1, ei1[0], ei1[1], zrows)
    return p0.reshape(2, NP, H), p1.reshape(2, NP, H)


def _pad_mlp(rs, l, W0, mlp_W1, mlp_b1, mlp_W2, mlp_b2, mlp_W3, mlp_b3, rank1):
    pads = HMP - HM
    W1s, b1s, W2s, b2s, W3s, b3s = [], [], [], [], [], []
    for r in rs:
        W1 = mlp_W1[l, r]
        if rank1:
            W1 = W0 @ W1
        W1s.append(jnp.pad(W1, ((0, 0), (0, pads))))
        b1s.append(jnp.pad(mlp_b1[l, r][None], ((0, 0), (0, pads))))
        W2s.append(jnp.pad(mlp_W2[l, r], ((0, pads), (0, pads))))
        b2s.append(jnp.pad(mlp_b2[l, r][None], ((0, 0), (0, pads))))
        W3s.append(jnp.pad(mlp_W3[l, r], ((0, pads), (0, 0))))
        b3s.append(mlp_b3[l, r][None])
    return (jnp.stack(W1s), jnp.stack(b1s), jnp.stack(W2s), jnp.stack(b2s),
            jnp.stack(W3s), jnp.stack(b3s))


def kernel(x_clause, x_variable, deg_clause, deg_variable, ei_cp, ei_cn,
           ei_rp, ei_rn, W0c, W0v, conv_ls_W, conv_ls_b, mlp_W1, mlp_b1,
           mlp_W2, mlp_b2, mlp_W3, mlp_b3, lins_c_W, lins_c_b, lins_v_W,
           lins_v_b):
    del conv_ls_W, conv_ls_b
    degc = deg_clause.reshape(N, 1)
    degv = deg_variable.reshape(N, 1)
    zrows = jnp.zeros((RPT, H), F32)

    wc = _pad_mlp((0, 1), 0, W0c, mlp_W1, mlp_b1, mlp_W2, mlp_b2, mlp_W3,
                  mlp_b3, rank1=True)
    zc0, zc1 = _mlp_pair(x_clause, degc, *wc, rank1=True)
    wv = _pad_mlp((2, 3), 0, W0v, mlp_W1, mlp_b1, mlp_W2, mlp_b2, mlp_W3,
                  mlp_b3, rank1=True)
    zv0, zv1 = _mlp_pair(x_variable, degv, *wv, rank1=True)

    pcp, pcn = _conv_pair(zv0, ei_rp, zv1, ei_rn, zrows)
    pvp, pvn = _conv_pair(zc0, ei_cp, zc1, ei_cn, zrows)

    xc1 = _combine(pcp, pcn, degc, x_clause,
                   lins_c_W[0, :H], lins_c_W[0, H:2 * H],
                   W0c @ lins_c_W[0, 2 * H:], lins_c_b[0][None], rank1=True)
    xv1 = _combine(pvp, pvn, degv, x_variable,
                   lins_v_W[0, :H], lins_v_W[0, H:2 * H],
                   W0v @ lins_v_W[0, 2 * H:], lins_v_b[0][None], rank1=True)

    wc1 = _pad_mlp((0, 1), 1, None, mlp_W1, mlp_b1, mlp_W2, mlp_b2, mlp_W3,
                   mlp_b3, rank1=False)
    zq0, zq1 = _mlp_pair(xc1, degc, *wc1, rank1=False)
    qvp, qvn = _conv_pair(zq0, ei_cp, zq1, ei_cn, zrows)

    xv2 = _combine(qvp, qvn, degv, xv1,
                   lins_v_W[1, :H], lins_v_W[1, H:2 * H],
                   lins_v_W[1, 2 * H:], lins_v_b[1][None], rank1=False)
    return xv2
